# initial kernel scaffold (unmeasured)
import jax
import jax.numpy as jnp
from jax import lax
from jax.experimental import pallas as pl
from jax.experimental.pallas import tpu as pltpu


def kernel(
    x,
):
    def body(*refs):
        pass

    out_shape = jax.ShapeDtypeStruct(..., jnp.float32)
    return pl.pallas_call(body, out_shape=out_shape)(...)


# baseline (device time: 2125406 ns/iter reference)
import jax
import jax.numpy as jnp
from jax import lax
from jax.experimental import pallas as pl
from jax.experimental.pallas import tpu as pltpu

NUM_CHUNKS = 16


def kernel(x):
    m_per, n = x.shape
    m_out = 2 * m_per
    half = m_per // 2
    chunk = half // NUM_CHUNKS

    def body(x_ref, out_ref, local_sem,
             x_send_sems, x_recv_sems, y_send_sems, y_recv_sems):
        my_x = lax.axis_index("x")
        my_y = lax.axis_index("y")
        x_nbr = (1 - my_x, my_y)
        y_nbr = (my_x, 1 - my_y)

        barrier_sem = pltpu.get_barrier_semaphore()
        for nbr in (x_nbr, y_nbr):
            pl.semaphore_signal(
                barrier_sem, inc=1,
                device_id=nbr, device_id_type=pl.DeviceIdType.MESH,
            )
        pl.semaphore_wait(barrier_sem, 2)

        local_copy = pltpu.make_async_copy(
            x_ref, out_ref.at[pl.ds(my_x * m_per, m_per), :], local_sem,
        )
        local_copy.start()

        send_base = my_x * m_per + my_y * half
        recv_base = (1 - my_x) * m_per + my_y * half

        x_rdmas = []
        for c in range(NUM_CHUNKS):
            r = pltpu.make_async_remote_copy(
                src_ref=x_ref.at[pl.ds(my_y * half + c * chunk, chunk), :],
                dst_ref=out_ref.at[pl.ds(send_base + c * chunk, chunk), :],
                send_sem=x_send_sems.at[c],
                recv_sem=x_recv_sems.at[c],
                device_id=x_nbr,
                device_id_type=pl.DeviceIdType.MESH,
            )
            r.start()
            x_rdmas.append(r)

        y_rdmas = []
        for c in range(NUM_CHUNKS):
            x_rdmas[c].wait_recv()
            r = pltpu.make_async_remote_copy(
                src_ref=out_ref.at[pl.ds(recv_base + c * chunk, chunk), :],
                dst_ref=out_ref.at[pl.ds(recv_base + c * chunk, chunk), :],
                send_sem=y_send_sems.at[c],
                recv_sem=y_recv_sems.at[c],
                device_id=y_nbr,
                device_id_type=pl.DeviceIdType.MESH,
            )
            r.start()
            y_rdmas.append(r)

        for c in range(NUM_CHUNKS):
            y_rdmas[c].wait_recv()
        for c in range(NUM_CHUNKS):
            x_rdmas[c].wait_send()
            y_rdmas[c].wait_send()
        local_copy.wait()

    return pl.pallas_call(
        body,
        out_shape=jax.ShapeDtypeStruct((m_out, n), x.dtype),
        in_specs=[pl.BlockSpec(memory_space=pl.ANY)],
        out_specs=pl.BlockSpec(memory_space=pl.ANY),
        scratch_shapes=[
            pltpu.SemaphoreType.DMA,
            pltpu.SemaphoreType.DMA((NUM_CHUNKS,)),
            pltpu.SemaphoreType.DMA((NUM_CHUNKS,)),
            pltpu.SemaphoreType.DMA((NUM_CHUNKS,)),
            pltpu.SemaphoreType.DMA((NUM_CHUNKS,)),
        ],
        compiler_params=pltpu.CompilerParams(collective_id=0),
    )(x)
